# baseline (speedup 1.0000x reference)
import jax
import jax.numpy as jnp
from jax import lax
from jax.experimental import pallas as pl
from jax.experimental.pallas import tpu as pltpu
from jax.experimental.pallas import tpu_sc as plsc

B = 16384
C = 2
N_IN = 234
N_OUT = 256
NUM_WORKERS = 32
KPW = N_OUT // NUM_WORKERS
CW = 2048
ROW_BYTES_ELEMS = C * B


def _sc_body(in_hbm, out_hbm, cbuf, rowbuf, sem):
    wid = lax.axis_index("s") * 2 + lax.axis_index("c")

    ones = jnp.ones((16,), jnp.float32)
    zeros = jnp.zeros((16,), jnp.float32)

    def fill(i, _):
        cbuf[0, pl.ds(i * 16, 16)] = ones
        cbuf[1, pl.ds(i * 16, 16)] = zeros
        cbuf[2, pl.ds(i * 16, 16)] = zeros
        cbuf[3, pl.ds(i * 16, 16)] = zeros
        return 0

    lax.fori_loop(0, CW // 16, fill, 0)

    for j in range(KPW):
        k = wid + NUM_WORKERS * j
        in_seg = (
            ((k >= 7) & (k <= 58))
            | ((k >= 60) & (k <= 124))
            | ((k >= 132) & (k <= 196))
            | ((k >= 198) & (k <= 249))
        )
        shift = (
            7
            + (k >= 60).astype(jnp.int32)
            + 7 * (k >= 132).astype(jnp.int32)
            + (k >= 198).astype(jnp.int32)
        )
        k_in = jnp.maximum(k - shift, 0)
        is_pilot = (k == 6) | (k == 59) | (k == 197) | (k == 250)
        r0 = jnp.where(is_pilot, 0, 2)

        @pl.when(in_seg)
        def _():
            pltpu.sync_copy(in_hbm.at[k_in], out_hbm.at[k])

        @pl.when(jnp.logical_not(in_seg))
        def _():
            for o in range(0, B, CW):
                pltpu.sync_copy(
                    cbuf.at[pl.ds(r0, 2), :],
                    out_hbm.at[k, :, pl.ds(o, CW)],
                )


def _sc_call(xT):
    mesh = plsc.VectorSubcoreMesh(core_axis_name="c", subcore_axis_name="s")
    return pl.kernel(
        _sc_body,
        out_type=jax.ShapeDtypeStruct((N_OUT, C, B), jnp.float32),
        mesh=mesh,
        scratch_types=[
            pltpu.VMEM((4, CW), jnp.float32),
            pltpu.VMEM((C, B), jnp.float32),
            pltpu.SemaphoreType.DMA,
        ],
    )(xT)


@jax.jit
def kernel(data_freq):
    xT = jnp.transpose(data_freq, (2, 1, 0))
    outT = _sc_call(xT)
    return jnp.transpose(outT, (2, 1, 0))


# speedup vs baseline: 13.1126x; 13.1126x over previous
import jax
import jax.numpy as jnp
from jax import lax
from jax.experimental import pallas as pl
from jax.experimental.pallas import tpu as pltpu
from jax.experimental.pallas import tpu_sc as plsc

B = 16384
C = 2
N_IN = 234
N_OUT = 256
NUM_WORKERS = 32
KPW = N_OUT // NUM_WORKERS
CW = 2048


def _classify(k):
    in_seg = (
        ((k >= 7) & (k <= 58))
        | ((k >= 60) & (k <= 124))
        | ((k >= 132) & (k <= 196))
        | ((k >= 198) & (k <= 249))
    )
    shift = (
        7
        + (k >= 60).astype(jnp.int32)
        + 7 * (k >= 132).astype(jnp.int32)
        + (k >= 198).astype(jnp.int32)
    )
    k_in = jnp.maximum(k - shift, 0)
    is_pilot = (k == 6) | (k == 59) | (k == 197) | (k == 250)
    r0 = jnp.where(is_pilot, 0, 2)
    return in_seg, k_in, r0


def _sc_body(in_hbm, out_hbm, cbuf, row_a, row_b, sia, sib, soa, sob):
    wid = lax.axis_index("s") * 2 + lax.axis_index("c")

    ones = jnp.ones((16,), jnp.float32)
    zeros = jnp.zeros((16,), jnp.float32)

    def fill(i, _):
        cbuf[0, pl.ds(i * 16, 16)] = ones
        cbuf[1, pl.ds(i * 16, 16)] = zeros
        cbuf[2, pl.ds(i * 16, 16)] = zeros
        cbuf[3, pl.ds(i * 16, 16)] = zeros
        return 0

    lax.fori_loop(0, CW // 16, fill, 0)

    bufs = ((row_a, sia, soa), (row_b, sib, sob))

    def out_descs(j, b):
        k = wid + NUM_WORKERS * j
        in_seg, _, r0 = _classify(k)
        rbuf, _, so = bufs[b]
        data_d = pltpu.make_async_copy(rbuf, out_hbm.at[k], so)
        const_d = [
            pltpu.make_async_copy(
                cbuf.at[pl.ds(r0, 2), :],
                out_hbm.at[k, :, pl.ds(o, CW)],
                so,
            )
            for o in range(0, B, CW)
        ]
        return in_seg, data_d, const_d

    def wait_out(j, b):
        in_seg, data_d, const_d = out_descs(j, b)

        @pl.when(in_seg)
        def _():
            data_d.wait()

        @pl.when(jnp.logical_not(in_seg))
        def _():
            for d in const_d:
                d.wait()

    for j in range(KPW):
        b = j & 1
        k = wid + NUM_WORKERS * j
        in_seg, k_in, _ = _classify(k)
        rbuf, si, _ = bufs[b]

        if j >= 2:
            wait_out(j - 2, b)

        @pl.when(in_seg)
        def _():
            pltpu.async_copy(in_hbm.at[k_in], rbuf, si)
            pltpu.make_async_copy(in_hbm.at[k_in], rbuf, si).wait()

        _, data_d, const_d = out_descs(j, b)

        @pl.when(in_seg)
        def _():
            data_d.start()

        @pl.when(jnp.logical_not(in_seg))
        def _():
            for d in const_d:
                d.start()

    wait_out(KPW - 2, 0)
    wait_out(KPW - 1, 1)


def _sc_call(xT):
    mesh = plsc.VectorSubcoreMesh(core_axis_name="c", subcore_axis_name="s")
    return pl.kernel(
        _sc_body,
        out_type=jax.ShapeDtypeStruct((N_OUT, C, B), jnp.float32),
        mesh=mesh,
        scratch_types=[
            pltpu.VMEM((4, CW), jnp.float32),
            pltpu.VMEM((C, B), jnp.float32),
            pltpu.VMEM((C, B), jnp.float32),
            pltpu.SemaphoreType.DMA,
            pltpu.SemaphoreType.DMA,
            pltpu.SemaphoreType.DMA,
            pltpu.SemaphoreType.DMA,
        ],
    )(xT)


@jax.jit
def kernel(data_freq):
    xT = jnp.transpose(data_freq, (2, 1, 0))
    outT = _sc_call(xT)
    return jnp.transpose(outT, (2, 1, 0))


# speedup vs baseline: 13.4642x; 1.0268x over previous
import jax
import jax.numpy as jnp
from jax import lax
from jax.experimental import pallas as pl
from jax.experimental.pallas import tpu as pltpu
from jax.experimental.pallas import tpu_sc as plsc

B = 16384
C = 2
N_IN = 234
N_OUT = 256
NUM_WORKERS = 32
KPW = N_OUT // NUM_WORKERS
HW = B // 2
NP = KPW * 2
CW = 2048


def _classify(k):
    in_seg = (
        ((k >= 7) & (k <= 58))
        | ((k >= 60) & (k <= 124))
        | ((k >= 132) & (k <= 196))
        | ((k >= 198) & (k <= 249))
    )
    shift = (
        7
        + (k >= 60).astype(jnp.int32)
        + 7 * (k >= 132).astype(jnp.int32)
        + (k >= 198).astype(jnp.int32)
    )
    k_in = jnp.maximum(k - shift, 0)
    is_pilot = (k == 6) | (k == 59) | (k == 197) | (k == 250)
    r0 = jnp.where(is_pilot, 0, 2)
    return in_seg, k_in, r0


def _sc_body(in_hbm, out_hbm, cbuf, h0, h1, h2, h3,
             si0, si1, si2, si3, so0, so1, so2, so3):
    wid = lax.axis_index("s") * 2 + lax.axis_index("c")

    ones = jnp.ones((16,), jnp.float32)
    zeros = jnp.zeros((16,), jnp.float32)

    def fill(i, _):
        cbuf[0, pl.ds(i * 16, 16)] = ones
        cbuf[1, pl.ds(i * 16, 16)] = zeros
        cbuf[2, pl.ds(i * 16, 16)] = zeros
        cbuf[3, pl.ds(i * 16, 16)] = zeros
        return 0

    lax.fori_loop(0, CW // 16, fill, 0)

    bufs = ((h0, si0, so0), (h1, si1, so1), (h2, si2, so2), (h3, si3, so3))

    def piece(p):
        j, h = p >> 1, p & 1
        k = wid + NUM_WORKERS * j
        in_seg, k_in, r0 = _classify(k)
        return k, h * HW, in_seg, k_in, r0

    def in_desc(p):
        k, o, in_seg, k_in, _ = piece(p)
        rbuf, si, _ = bufs[p & 3]
        return in_seg, pltpu.make_async_copy(
            in_hbm.at[k_in, :, pl.ds(o, HW)], rbuf, si)

    def out_descs(p):
        k, o, in_seg, _, r0 = piece(p)
        rbuf, _, so = bufs[p & 3]
        data_d = pltpu.make_async_copy(
            rbuf, out_hbm.at[k, :, pl.ds(o, HW)], so)
        const_d = [
            pltpu.make_async_copy(
                cbuf.at[pl.ds(r0, 2), :],
                out_hbm.at[k, :, pl.ds(o + q, CW)],
                so,
            )
            for q in range(0, HW, CW)
        ]
        return in_seg, data_d, const_d

    def start_in(p):
        in_seg, d = in_desc(p)

        @pl.when(in_seg)
        def _():
            d.start()

    def wait_in(p):
        in_seg, d = in_desc(p)

        @pl.when(in_seg)
        def _():
            d.wait()

    def start_out(p):
        in_seg, data_d, const_d = out_descs(p)

        @pl.when(in_seg)
        def _():
            data_d.start()

        @pl.when(jnp.logical_not(in_seg))
        def _():
            for d in const_d:
                d.start()

    def wait_out(p):
        in_seg, data_d, const_d = out_descs(p)

        @pl.when(in_seg)
        def _():
            data_d.wait()

        @pl.when(jnp.logical_not(in_seg))
        def _():
            for d in const_d:
                d.wait()

    start_in(0)
    start_in(1)
    for p in range(NP):
        if p >= 2:
            wait_out(p - 2)
        wait_in(p)
        start_out(p)
        if p + 2 < NP:
            start_in(p + 2)
    wait_out(NP - 2)
    wait_out(NP - 1)


def _sc_call(xT):
    mesh = plsc.VectorSubcoreMesh(core_axis_name="c", subcore_axis_name="s")
    return pl.kernel(
        _sc_body,
        out_type=jax.ShapeDtypeStruct((N_OUT, C, B), jnp.float32),
        mesh=mesh,
        scratch_types=[
            pltpu.VMEM((4, CW), jnp.float32),
            pltpu.VMEM((C, HW), jnp.float32),
            pltpu.VMEM((C, HW), jnp.float32),
            pltpu.VMEM((C, HW), jnp.float32),
            pltpu.VMEM((C, HW), jnp.float32),
            pltpu.SemaphoreType.DMA,
            pltpu.SemaphoreType.DMA,
            pltpu.SemaphoreType.DMA,
            pltpu.SemaphoreType.DMA,
            pltpu.SemaphoreType.DMA,
            pltpu.SemaphoreType.DMA,
            pltpu.SemaphoreType.DMA,
            pltpu.SemaphoreType.DMA,
        ],
    )(xT)


@jax.jit
def kernel(data_freq):
    xT = jnp.transpose(data_freq, (2, 1, 0))
    outT = _sc_call(xT)
    return jnp.transpose(outT, (2, 1, 0))
